# baseline (speedup 1.0000x reference)
import functools
import jax
import jax.numpy as jnp
from jax import lax
from jax.experimental import pallas as pl
from jax.experimental.pallas import tpu as pltpu
from jax.experimental.pallas import tpu_sc as plsc

NEG = float('-inf')


def _conv(x, w, b, stride=1, pad=1):
    y = lax.conv_general_dilated(
        x, w, (stride, stride), [(pad, pad), (pad, pad)],
        dimension_numbers=('NCHW', 'OIHW', 'NCHW'))
    return y + b[None, :, None, None]


def _shift_rows(a, s):
    f = jnp.full((abs(s), a.shape[1]), NEG, a.dtype)
    if s > 0:
        return jnp.concatenate([a[s:], f], 0)
    return jnp.concatenate([f, a[:s]], 0)


def _shift_cols(a, s):
    f = jnp.full((a.shape[0], abs(s)), NEG, a.dtype)
    if s > 0:
        return jnp.concatenate([a[:, s:], f], 1)
    return jnp.concatenate([f, a[:, :s]], 1)


def _wsel(av, ai, bv, bi):
    p = (av > bv) | ((av == bv) & (ai < bi))
    return p, jnp.where(p, av, bv), jnp.where(p, ai, bi), jnp.where(p, bv, av), jnp.where(p, bi, ai)


def _nms_kernel(r_ref, v1_ref, i1_ref, v2_ref, i2_ref):
    m = r_ref[0]
    rm = m
    for s in (-2, -1, 1, 2):
        rm = jnp.maximum(rm, _shift_rows(m, s))
    mp = rm
    for s in (-2, -1, 1, 2):
        mp = jnp.maximum(mp, _shift_cols(rm, s))
    rows = lax.broadcasted_iota(jnp.int32, (512, 512), 0)
    cols = lax.broadcasted_iota(jnp.int32, (512, 512), 1)
    border = (rows >= 4) & (rows < 508) & (cols >= 4) & (cols < 508)
    det = (m == mp) & (m > 0.0) & border
    mv = jnp.where(det, m, NEG)
    idx = rows * 512 + cols
    mv4 = mv.reshape(256, 2, 512)
    id4 = idx.reshape(256, 2, 512)
    _, wv, wi, lv, li = _wsel(mv4[:, 0], id4[:, 0], mv4[:, 1], id4[:, 1])
    wvT = wv.T.reshape(256, 2, 256)
    wiT = wi.T.reshape(256, 2, 256)
    lvT = lv.T.reshape(256, 2, 256)
    liT = li.T.reshape(256, 2, 256)
    p, fv, fi, s0v, s0i = _wsel(wvT[:, 0], wiT[:, 0], wvT[:, 1], wiT[:, 1])
    altv = jnp.where(p, lvT[:, 0], lvT[:, 1])
    alti = jnp.where(p, liT[:, 0], liT[:, 1])
    _, sv, si, _, _ = _wsel(s0v, s0i, altv, alti)
    v1_ref[0] = fv
    i1_ref[0] = fi
    v2_ref[0] = sv
    i2_ref[0] = si


def _nms_cells(r2):
    return pl.pallas_call(
        _nms_kernel,
        grid=(4,),
        in_specs=[pl.BlockSpec((1, 512, 512), lambda b: (b, 0, 0))],
        out_specs=[pl.BlockSpec((1, 256, 256), lambda b: (b, 0, 0))] * 4,
        out_shape=[
            jax.ShapeDtypeStruct((4, 256, 256), jnp.float32),
            jax.ShapeDtypeStruct((4, 256, 256), jnp.int32),
            jax.ShapeDtypeStruct((4, 256, 256), jnp.float32),
            jax.ShapeDtypeStruct((4, 256, 256), jnp.int32),
        ],
    )(r2)


def _stage_rows(v, i, J, lev):
    G = 1024 // (2 * J)
    D = 2 ** (lev - 7)
    v4 = v.reshape(G, 2, J, 128)
    i4 = i.reshape(G, 2, J, 128)
    av, bv, ai, bi = v4[:, 0], v4[:, 1], i4[:, 0], i4[:, 1]
    dm = (lax.broadcasted_iota(jnp.int32, (G, J, 128), 0) // max(1, D // (2 * J))) % 2 == 0
    p = (av > bv) | ((av == bv) & (ai < bi))
    sw = p != dm
    nav = jnp.where(sw, bv, av)
    nbv = jnp.where(sw, av, bv)
    nai = jnp.where(sw, bi, ai)
    nbi = jnp.where(sw, ai, bi)
    v = jnp.stack([nav, nbv], 1).reshape(1024, 128)
    i = jnp.stack([nai, nbi], 1).reshape(1024, 128)
    return v, i


def _stage_lanes(vw, iw, j, lev):
    G = 128 // (2 * j)
    v4 = vw.reshape(G, 2, j, 1024)
    i4 = iw.reshape(G, 2, j, 1024)
    av, bv, ai, bi = v4[:, 0], v4[:, 1], i4[:, 0], i4[:, 1]
    if lev < 7:
        dm = (lax.broadcasted_iota(jnp.int32, (G, j, 1024), 0) // max(1, (2 ** lev) // (2 * j))) % 2 == 0
    else:
        dm = (lax.broadcasted_iota(jnp.int32, (G, j, 1024), 2) // (2 ** (lev - 7))) % 2 == 0
    p = (av > bv) | ((av == bv) & (ai < bi))
    sw = p != dm
    nav = jnp.where(sw, bv, av)
    nbv = jnp.where(sw, av, bv)
    nai = jnp.where(sw, bi, ai)
    nbi = jnp.where(sw, ai, bi)
    vw = jnp.stack([nav, nbv], 1).reshape(128, 1024)
    iw = jnp.stack([nai, nbi], 1).reshape(128, 1024)
    return vw, iw


def _sort_kernel(v_ref, i_ref, s_ref, xs_ref, ys_ref,
                 g00_ref, g01_ref, g10_ref, g11_ref,
                 w00_ref, w01_ref, w10_ref, w11_ref):
    b = pl.program_id(0)
    v = v_ref[0]
    i = i_ref[0]
    vw = v.T
    iw = i.T
    in_w = True
    for lev in range(1, 18):
        for k in range(lev - 1, -1, -1):
            j = 2 ** k
            if j >= 128:
                if in_w:
                    v, i = vw.T, iw.T
                    in_w = False
                v, i = _stage_rows(v, i, j // 128, lev)
            else:
                if not in_w:
                    vw, iw = v.T, i.T
                    in_w = True
                vw, iw = _stage_lanes(vw, iw, j, lev)
    if in_w:
        v, i = vw.T, iw.T

    sv = v[:32]
    si = i[:32]
    npos = jnp.sum((sv > 0.0).astype(jnp.int32))
    rank = lax.broadcasted_iota(jnp.int32, (32, 128), 0) * 128 + lax.broadcasted_iota(jnp.int32, (32, 128), 1)
    oi = jnp.where(sv > 0.0, si, rank - npos)
    os = jnp.where(sv > 0.0, sv, NEG)
    xs = (oi & 511).astype(jnp.float32)
    ys = (oi >> 9).astype(jnp.float32)
    s_ref[0] = os
    xs_ref[0] = xs
    ys_ref[0] = ys

    def corner(coord):
        g = (coord + 0.5) / 512.0 * 2.0 - 1.0
        ic = ((g + 1.0) * 64.0 - 1.0) / 2.0
        c0 = jnp.floor(ic)
        c1 = c0 + 1.0
        f1 = ic - c0
        f0 = 1.0 - f1
        v0 = (c0 >= 0) & (c0 <= 63)
        v1 = (c1 >= 0) & (c1 <= 63)
        cc0 = jnp.clip(c0, 0, 63).astype(jnp.int32)
        cc1 = jnp.clip(c1, 0, 63).astype(jnp.int32)
        return cc0, cc1, f0, f1, v0, v1

    cx0, cx1, fx0, fx1, vx0, vx1 = corner(xs)
    cy0, cy1, fy0, fy1, vy0, vy1 = corner(ys)
    base = b * 4096
    g00_ref[0] = base + cy0 * 64 + cx0
    g01_ref[0] = base + cy0 * 64 + cx1
    g10_ref[0] = base + cy1 * 64 + cx0
    g11_ref[0] = base + cy1 * 64 + cx1
    w00_ref[0] = (fx0 * fy0) * (vx0 & vy0).astype(jnp.float32)
    w01_ref[0] = (fx1 * fy0) * (vx1 & vy0).astype(jnp.float32)
    w10_ref[0] = (fx0 * fy1) * (vx0 & vy1).astype(jnp.float32)
    w11_ref[0] = (fx1 * fy1) * (vx1 & vy1).astype(jnp.float32)


def _sort_topk(cv, ci):
    f32 = jnp.float32
    i32 = jnp.int32
    outs = [f32, f32, f32, i32, i32, i32, i32, f32, f32, f32, f32]
    return pl.pallas_call(
        _sort_kernel,
        grid=(4,),
        in_specs=[pl.BlockSpec((1, 1024, 128), lambda b: (b, 0, 0))] * 2,
        out_specs=[pl.BlockSpec((1, 32, 128), lambda b: (b, 0, 0))] * 11,
        out_shape=[jax.ShapeDtypeStruct((4, 32, 128), d) for d in outs],
    )(cv, ci)


def _sc_gather_body(table_hbm, gidx_hbm, out_hbm, idx_v, buf_v, sem):
    info = plsc.get_sparse_core_info()
    nc = info.num_cores
    wid = lax.axis_index("s") * nc + lax.axis_index("c")
    pltpu.sync_copy(gidx_hbm.at[pl.ds(wid * 16, 16)], idx_v)
    cps = [pltpu.async_copy(table_hbm.at[idx_v.at[k]], buf_v.at[k], sem)
           for k in range(16)]
    for cp in cps:
        cp.wait()
    pltpu.sync_copy(buf_v, out_hbm.at[pl.ds(wid * 16, 16)])


@functools.partial(jax.jit, static_argnums=())
def _sc_gather(table, gidx):
    mesh = plsc.VectorSubcoreMesh(core_axis_name="c", subcore_axis_name="s")
    k = functools.partial(
        pl.kernel,
        mesh=mesh,
        compiler_params=pltpu.CompilerParams(use_tc_tiling_on_sc=False),
        out_type=jax.ShapeDtypeStruct((512, 128, 32), jnp.float32),
        scratch_types=[
            pltpu.VMEM((16, 128), jnp.int32),
            pltpu.VMEM((16, 128, 32), jnp.float32),
            pltpu.SemaphoreType.DMA,
        ],
    )(_sc_gather_body)
    return k(table, gidx)


def _combine_kernel(g_ref, w_ref, o_ref):
    t = [g_ref[c] * jnp.broadcast_to(w_ref[c], (32, 16384)) for c in range(4)]
    o_ref[...] = ((t[0] + t[1]) + t[2]) + t[3]


def _combine(gathT, w4):
    return pl.pallas_call(
        _combine_kernel,
        out_shape=jax.ShapeDtypeStruct((32, 16384), jnp.float32),
    )(gathT, w4)


def kernel(x, w1, b1, w2, b2, w3, b3, w4, b4, wd, bd, wt1, bt1, wt2, bt2):
    f1 = jax.nn.relu(_conv(x, w1, b1, 1, 1))
    f2 = jax.nn.relu(_conv(f1, w2, b2, 2, 1))
    f3 = jax.nn.relu(_conv(f2, w3, b3, 2, 1))
    f4 = jax.nn.relu(_conv(f3, w4, b4, 2, 1))
    raw_desc = _conv(f4, wd, bd, 1, 0)
    t = jax.nn.relu(_conv(f1, wt1, bt1, 1, 1))
    raw_detect = _conv(t, wt2, bt2, 1, 1)

    fv, fi, sv, si = _nms_cells(raw_detect[:, 0])
    cv = jnp.concatenate([fv.reshape(4, 512, 128), sv.reshape(4, 512, 128)], 1)
    ci = jnp.concatenate([fi.reshape(4, 512, 128), si.reshape(4, 512, 128)], 1)
    s, xs, ys, g00, g01, g10, g11, w00, w01, w10, w11 = _sort_topk(cv, ci)

    scores = s.reshape(4, 4096)
    kpts = jnp.stack([xs.reshape(4, 4096), ys.reshape(4, 4096)], -1)

    table = raw_desc.transpose(0, 2, 3, 1).reshape(16384, 32)
    gidx = jnp.stack([g00, g01, g10, g11], 0).reshape(512, 128)
    gath = _sc_gather(table, gidx)
    gathT = gath.reshape(4, 16384, 32).transpose(0, 2, 1)
    w4a = jnp.stack([w00, w01, w10, w11], 0).reshape(4, 1, 16384)
    descsT = _combine(gathT, w4a)
    descs = descsT.T.reshape(4, 4096, 32)
    return kpts, scores, descs
